# baseline (speedup 1.0000x reference)
import jax
import jax.numpy as jnp
from jax.experimental import pallas as pl
from jax.experimental.pallas import tpu as pltpu

N = 2048
HID = 128
NH = 4
HD = 32
K = 32
EC = 32768
RB = 256
NB = N // RB


def _block_kernel(src_ref, col_ref, wgt_ref, src_v_ref, e_blk_ref, e_t_ref, w_ref,
                  m_l_ref, m_r_ref, idx_out_ref, w_out_ref,
                  base_s, work_s, starts_s):
    pid = pl.program_id(0)

    @pl.when(pid == 0)
    def _():
        bid = src_v_ref[...] >> 8
        starts_s[0] = 0
        for b in range(1, NB + 1):
            starts_s[b] = jnp.sum((bid < b).astype(jnp.int32))

    h_blk = jnp.dot(e_blk_ref[...], w_ref[...], preferred_element_type=jnp.float32)
    s_i = jnp.dot(h_blk, m_l_ref[...], preferred_element_type=jnp.float32)
    h_t = jax.lax.dot_general(w_ref[...], e_t_ref[...],
                              (((0,), (0,)), ((), ())),
                              preferred_element_type=jnp.float32)
    s_r_t = jax.lax.dot_general(m_r_ref[...], h_t,
                                (((0,), (0,)), ((), ())),
                                preferred_element_type=jnp.float32)
    NC = N // 128
    for ch in range(NC):
        sl = slice(ch * 128, (ch + 1) * 128)
        acc = None
        for hh in range(NH):
            x = s_i[:, hh:hh + 1] + s_r_t[hh:hh + 1, sl]
            x = jnp.where(x >= 0, x, 0.2 * x)
            acc = x if acc is None else acc + x
        b = acc * 0.25
        base_s[:, ch, :] = b
        work_s[:, ch, :] = b

    e0 = starts_s[pid]
    e1 = starts_s[pid + 1]
    row0 = pid * RB
    lane_iota = jax.lax.broadcasted_iota(jnp.int32, (1, 1, 128), 2)

    def body(e, carry):
        r = src_ref[0, e] - row0
        c = col_ref[0, e]
        ch = c // 128
        lane = c - ch * 128
        bslab = base_s[pl.ds(r, 1), pl.ds(ch, 1), :]
        wslab = work_s[pl.ds(r, 1), pl.ds(ch, 1), :]
        work_s[pl.ds(r, 1), pl.ds(ch, 1), :] = jnp.where(
            lane_iota == lane, bslab + wgt_ref[0, e], wslab)
        return carry

    jax.lax.fori_loop(e0, e1, body, 0)

    colid = (jax.lax.broadcasted_iota(jnp.int32, (RB, NC, 128), 1) * 128
             + jax.lax.broadcasted_iota(jnp.int32, (RB, NC, 128), 2))
    work = work_s[...]
    vals = []
    idxs = []
    for t in range(K):
        m = jnp.max(jnp.max(work, axis=2), axis=1)
        is_max = work == m[:, None, None]
        idx = jnp.min(jnp.min(jnp.where(is_max, colid, N), axis=2), axis=1)
        vals.append(m[:, None])
        idxs.append(idx[:, None])
        if t + 1 < K:
            work = jnp.where(colid == idx[:, None, None], -jnp.inf, work)
    vals_m = jnp.concatenate(vals, axis=1)
    idx_out_ref[...] = jnp.concatenate(idxs, axis=1)

    ex = jnp.exp(vals_m - vals_m[:, 0:1])
    w_out_ref[...] = ex / jnp.sum(ex, axis=1, keepdims=True)


def kernel(embeddings, corr_edge_index, corr_edge_weight, return_weights, W, att, corr_lambda):
    n = embeddings.shape[0]
    a_l = att[:, :HD]
    a_r = att[:, HD:]
    eye = jnp.eye(NH, dtype=jnp.float32)
    m_l = (a_l[:, :, None] * eye[:, None, :]).reshape(HID, NH)
    m_r = (a_r[:, :, None] * eye[:, None, :]).reshape(HID, NH)

    src = corr_edge_index[0]
    col = corr_edge_index[1]
    perm = jnp.argsort(src // RB, stable=True)
    src_s = src[perm].reshape(1, EC)
    col_s = col[perm].reshape(1, EC)
    wgt_s = (corr_lambda[0] * corr_edge_weight[perm]).reshape(1, EC)

    grid = (NB,)
    in_specs = [
            pl.BlockSpec(memory_space=pltpu.SMEM),
            pl.BlockSpec(memory_space=pltpu.SMEM),
            pl.BlockSpec(memory_space=pltpu.SMEM),
            pl.BlockSpec((1, EC), lambda b: (0, 0)),
            pl.BlockSpec((RB, HID), lambda b: (b, 0)),
            pl.BlockSpec((HID, N), lambda b: (0, 0)),
            pl.BlockSpec((HID, HID), lambda b: (0, 0)),
            pl.BlockSpec((HID, NH), lambda b: (0, 0)),
            pl.BlockSpec((HID, NH), lambda b: (0, 0)),
    ]
    out_specs = [
            pl.BlockSpec((RB, K), lambda b: (b, 0)),
            pl.BlockSpec((RB, K), lambda b: (b, 0)),
    ]
    idx_mat, w_mat = pl.pallas_call(
        _block_kernel,
        grid=grid,
        in_specs=in_specs,
        out_specs=out_specs,
        out_shape=[
            jax.ShapeDtypeStruct((n, K), jnp.int32),
            jax.ShapeDtypeStruct((n, K), jnp.float32),
        ],
        scratch_shapes=[
            pltpu.VMEM((RB, N // 128, 128), jnp.float32),
            pltpu.VMEM((RB, N // 128, 128), jnp.float32),
            pltpu.SMEM((NB + 1,), jnp.int32),
        ],
    )(src_s, col_s, wgt_s, src_s, embeddings, embeddings.T, W, m_l, m_r)

    row_idx = jnp.broadcast_to(jnp.arange(n, dtype=idx_mat.dtype)[:, None], (n, K))
    edge_index = jnp.stack([idx_mat.reshape(-1), row_idx.reshape(-1)], axis=0)
    gate = (jnp.asarray(return_weights) != 0).astype(jnp.float32)
    edge_weight = gate * w_mat.reshape(-1)
    return (edge_index, edge_weight)
